# Initial kernel scaffold; baseline (speedup 1.0000x reference)
import jax
import jax.numpy as jnp
from jax.experimental import pallas as pl


def kernel(x, logits_t, logits_u, sigma_min, sigma_max):
    raise NotImplementedError("write your pallas kernel here")


# speedup vs baseline: 1610.6101x; 1610.6101x over previous
import functools
import jax
import jax.numpy as jnp
from jax import lax
from jax.experimental import pallas as pl

F = 100
BINS = 100
BLK = 2048


def _tables_body(lt_ref, lu_ref, smin_ref, smax_ref, ds_ref, l_ref, c0_ref):
    lt = lt_ref[...]
    lu = lu_ref[...]
    smin = smin_ref[...]
    smax = smax_ref[...]
    wu = jnp.exp(lu) + 1e-7
    wt = jax.nn.softmax(lt, axis=1) + 1e-7
    wt = wt / jnp.sum(wt, axis=1, keepdims=True)
    s = wu / wt
    row = lax.broadcasted_iota(jnp.int32, (BINS, BINS), 0)
    col = lax.broadcasted_iota(jnp.int32, (BINS, BINS), 1)
    tri = (row < col).astype(jnp.float32)
    lt_left = jnp.dot(wt, tri, preferred_element_type=jnp.float32)
    ds = s - jnp.concatenate([jnp.zeros((F, 1), jnp.float32), s[:, :-1]], axis=1)
    rng = smax - smin
    ds_scaled = ds / rng
    l_scaled = smin + lt_left * rng
    ds_ref[...] = ds_scaled
    l_ref[...] = l_scaled
    c0_ref[...] = jnp.sum(ds_scaled * l_scaled, axis=1, keepdims=True)


def _build_tables(logits_t, logits_u, sigma_min, sigma_max):
    smin = sigma_min.reshape(F, 1)
    smax = sigma_max.reshape(F, 1)
    return pl.pallas_call(
        _tables_body,
        out_shape=(
            jax.ShapeDtypeStruct((F, BINS), jnp.float32),
            jax.ShapeDtypeStruct((F, BINS), jnp.float32),
            jax.ShapeDtypeStruct((F, 1), jnp.float32),
        ),
    )(logits_t, logits_u, smin, smax)


def _sweep_body(x_ref, ds_ref, l_ref, c0_ref, o_ref):
    xb = x_ref[...]
    acc = jnp.zeros_like(xb)
    for k in range(BINS):
        lrow = l_ref[k:k + 1, :]
        dsrow = ds_ref[k:k + 1, :]
        acc = acc + dsrow * jnp.maximum(xb, lrow)
    o_ref[...] = acc - c0_ref[0:1, :]


@jax.jit
def kernel(x, logits_t, logits_u, sigma_min, sigma_max):
    ds, l, c0 = _build_tables(logits_t, logits_u, sigma_min, sigma_max)
    ds_t = ds.T
    l_t = l.T
    c0_t = c0.T
    b = x.shape[0]
    grid = b // BLK
    return pl.pallas_call(
        _sweep_body,
        grid=(grid,),
        in_specs=[
            pl.BlockSpec((BLK, F), lambda i: (i, 0)),
            pl.BlockSpec((BINS, F), lambda i: (0, 0)),
            pl.BlockSpec((BINS, F), lambda i: (0, 0)),
            pl.BlockSpec((1, F), lambda i: (0, 0)),
        ],
        out_specs=pl.BlockSpec((BLK, F), lambda i: (i, 0)),
        out_shape=jax.ShapeDtypeStruct((b, F), jnp.float32),
    )(x, ds_t, l_t, c0_t)
